# Initial kernel scaffold; baseline (speedup 1.0000x reference)
import jax
import jax.numpy as jnp
from jax.experimental import pallas as pl


def kernel(x, adj_t, adj_t2):
    raise NotImplementedError("write your pallas kernel here")


# speedup vs baseline: 4.4520x; 4.4520x over previous
import functools

import jax
import jax.numpy as jnp
from jax import lax
from jax.experimental import pallas as pl
from jax.experimental.pallas import tpu as pltpu
from jax.experimental.pallas import tpu_sc as plsc

D = 128
CHUNK = 80
NSUB = 16
EDGE_ALIGN = NSUB * CHUNK


@functools.lru_cache(maxsize=None)
def _make_sc_kernel(n, e1, e2):
    mesh = plsc.VectorSubcoreMesh(core_axis_name="c", subcore_axis_name="s")
    per_tile1 = e1 // NSUB
    per_tile2 = e2 // NSUB
    n_iter1 = per_tile1 // CHUNK
    n_iter2 = per_tile2 // CHUNK
    rows_out = -(-n // (NSUB * 8)) * 8
    n_acc = rows_out * NSUB

    @functools.partial(
        pl.kernel,
        mesh=mesh,
        out_type=[jax.ShapeDtypeStruct((n_acc, D), jnp.float32),
                  jax.ShapeDtypeStruct((n_acc, D), jnp.float32)],
        scratch_types=[
            pltpu.VMEM((CHUNK,), jnp.int32),
            pltpu.VMEM((CHUNK,), jnp.int32),
            pltpu.VMEM((CHUNK, D), jnp.float32),
            pltpu.VMEM_SHARED((n_acc, D), jnp.float32),
        ],
    )
    def k(x_hbm, s1_hbm, d1_hbm, s2_hbm, d2_hbm, z_hbm, o1_hbm, o2_hbm,
          src_v, dst_v, rows_v, acc):
        sid = lax.axis_index("s")
        cid = lax.axis_index("c")

        pltpu.sync_copy(z_hbm.at[pl.ds(sid * rows_out, rows_out)],
                        acc.at[pl.ds(sid * rows_out, rows_out)])

        plsc.subcore_barrier()

        def process(s_hbm, d_hbm, per_tile, n_iter):
            base = sid * per_tile

            @pl.loop(0, n_iter)
            def _(t):
                e0 = base + t * CHUNK
                pltpu.sync_copy(s_hbm.at[pl.ds(e0, CHUNK)], src_v)
                pltpu.sync_copy(d_hbm.at[pl.ds(e0, CHUNK)], dst_v)
                pltpu.sync_copy(x_hbm.at[src_v], rows_v)
                pltpu.sync_copy(rows_v, acc.at[dst_v], add=True)

        @pl.when(cid == 0)
        def _():
            process(s1_hbm, d1_hbm, per_tile1, n_iter1)

        @pl.when(cid == 1)
        def _():
            process(s2_hbm, d2_hbm, per_tile2, n_iter2)

        plsc.subcore_barrier()

        @pl.when(cid == 0)
        def _():
            pltpu.sync_copy(acc.at[pl.ds(sid * rows_out, rows_out)],
                            o1_hbm.at[pl.ds(sid * rows_out, rows_out)])

        @pl.when(cid == 1)
        def _():
            pltpu.sync_copy(acc.at[pl.ds(sid * rows_out, rows_out)],
                            o2_hbm.at[pl.ds(sid * rows_out, rows_out)])

    return k


def _pad_edges(src, dst, n):
    e = src.shape[0]
    e_pad = -(-e // EDGE_ALIGN) * EDGE_ALIGN
    if e_pad != e:
        pad = e_pad - e
        src = jnp.concatenate([src, jnp.zeros((pad,), src.dtype)])
        dst = jnp.concatenate([dst, jnp.full((pad,), n, dst.dtype)])
    return src, dst


def kernel(x, adj_t, adj_t2):
    n = x.shape[0]
    s1, d1 = _pad_edges(adj_t[0], adj_t[1], n)
    s2, d2 = _pad_edges(adj_t2[0], adj_t2[1], n)
    n_acc = -(-n // (NSUB * 8)) * 8 * NSUB
    zeros = jnp.zeros((n_acc, D), jnp.float32)
    k = _make_sc_kernel(n, s1.shape[0], s2.shape[0])
    x1, x2 = k(x, s1, d1, s2, d2, zeros)
    return jnp.concatenate([x1[:n], x2[:n]], axis=1)


# speedup vs baseline: 6.9331x; 1.5573x over previous
import functools

import jax
import jax.numpy as jnp
from jax import lax
from jax.experimental import pallas as pl
from jax.experimental.pallas import tpu as pltpu
from jax.experimental.pallas import tpu_sc as plsc

D = 128
CHUNK = 80
NSUB = 16
NB = 4
EDGE_ALIGN = NSUB * CHUNK * 2 * NB


@functools.lru_cache(maxsize=None)
def _make_sc_kernel(n, e1, e2):
    mesh = plsc.VectorSubcoreMesh(core_axis_name="c", subcore_axis_name="s")
    per_tile1 = e1 // NSUB
    per_tile2 = e2 // NSUB
    n_iter1 = per_tile1 // CHUNK
    n_iter2 = per_tile2 // CHUNK
    rows_out = -(-n // (NSUB * 8)) * 8
    n_acc = rows_out * NSUB

    out_sds = jax.ShapeDtypeStruct((n_acc, D), jnp.float32)

    @functools.partial(
        pl.kernel,
        mesh=mesh,
        out_type=[out_sds, out_sds],
        scratch_types=[
            pltpu.VMEM((2 * NB, CHUNK), jnp.int32),
            pltpu.VMEM((2 * NB, CHUNK), jnp.int32),
            pltpu.VMEM((NB, CHUNK, D), jnp.float32),
            pltpu.VMEM_SHARED((n_acc, D), jnp.float32),
            pltpu.SemaphoreType.DMA((2 * NB,)),
            pltpu.SemaphoreType.DMA((NB,)),
            pltpu.SemaphoreType.DMA((NB,)),
        ],
    )
    def k(x_hbm, s1_hbm, d1_hbm, s2_hbm, d2_hbm, z_hbm, o1_hbm, o2_hbm,
          src_v, dst_v, rows_v, acc, isem, gsem, ssem):
        sid = lax.axis_index("s")
        cid = lax.axis_index("c")

        row0 = sid * rows_out
        pltpu.sync_copy(z_hbm.at[pl.ds(row0, rows_out)],
                        acc.at[pl.ds(row0, rows_out)])
        plsc.subcore_barrier()

        def process(s_hbm, d_hbm, per_tile, n_iter):
            base = sid * per_tile

            def idx_start(c, u):
                e0 = base + c * CHUNK
                pltpu.async_copy(s_hbm.at[pl.ds(e0, CHUNK)],
                                 src_v.at[u], isem.at[u])
                pltpu.async_copy(d_hbm.at[pl.ds(e0, CHUNK)],
                                 dst_v.at[u], isem.at[u])

            def idx_wait(u):
                pltpu.make_async_copy(s_hbm.at[pl.ds(0, CHUNK)],
                                      src_v.at[u], isem.at[u]).wait()
                pltpu.make_async_copy(d_hbm.at[pl.ds(0, CHUNK)],
                                      dst_v.at[u], isem.at[u]).wait()

            def gather_wait(b):
                pltpu.make_async_copy(z_hbm.at[pl.ds(0, CHUNK)],
                                      rows_v.at[b], gsem.at[b]).wait()

            def scatter_wait(b):
                pltpu.make_async_copy(z_hbm.at[pl.ds(0, CHUNK)],
                                      rows_v.at[b], ssem.at[b]).wait()

            for b in range(NB):
                idx_start(b, b)

            @pl.loop(0, n_iter, step=2 * NB)
            def _(r):
                for S in (0, 1):
                    cb = r + S * NB
                    for b in range(NB):
                        u = S * NB + b
                        idx_wait(u)
                        if S == 0:
                            @pl.when(r > 0)
                            def _():
                                scatter_wait(b)
                        else:
                            scatter_wait(b)

                        pltpu.async_copy(x_hbm.at[src_v.at[u]],
                                         rows_v.at[b], gsem.at[b])

                    for b in range(NB):
                        c2 = cb + NB + b
                        u2 = (1 - S) * NB + b

                        @pl.when(c2 < n_iter)
                        def _():
                            idx_start(c2, u2)

                    for b in range(NB):
                        u = S * NB + b
                        gather_wait(b)
                        pltpu.async_copy(rows_v.at[b], acc.at[dst_v.at[u]],
                                         ssem.at[b], add=True)

            for b in range(NB):
                scatter_wait(b)

        @pl.when(cid == 0)
        def _():
            process(s1_hbm, d1_hbm, per_tile1, n_iter1)

        @pl.when(cid == 1)
        def _():
            process(s2_hbm, d2_hbm, per_tile2, n_iter2)

        plsc.subcore_barrier()

        sl = pl.ds(row0, rows_out)

        @pl.when(cid == 0)
        def _():
            pltpu.sync_copy(acc.at[sl], o1_hbm.at[sl])

        @pl.when(cid == 1)
        def _():
            pltpu.sync_copy(acc.at[sl], o2_hbm.at[sl])

    return k


def _pad_edges(src, dst, n, n_acc):
    e = src.shape[0]
    e_pad = -(-e // EDGE_ALIGN) * EDGE_ALIGN
    if e_pad != e:
        pad = e_pad - e
        src = jnp.concatenate([src, jnp.zeros((pad,), src.dtype)])
        trash = n + jnp.arange(pad, dtype=dst.dtype) % (n_acc - n)
        dst = jnp.concatenate([dst, trash])
    return src, dst


def kernel(x, adj_t, adj_t2):
    n = x.shape[0]
    n_acc = -(-n // (NSUB * 8)) * 8 * NSUB
    s1, d1 = _pad_edges(adj_t[0], adj_t[1], n, n_acc)
    s2, d2 = _pad_edges(adj_t2[0], adj_t2[1], n, n_acc)
    zeros = jnp.zeros((n_acc, D), jnp.float32)
    k = _make_sc_kernel(n, s1.shape[0], s2.shape[0])
    x1, x2 = k(x, s1, d1, s2, d2, zeros)
    return jnp.concatenate([x1[:n], x2[:n]], axis=1)


# speedup vs baseline: 9.4317x; 1.3604x over previous
import functools

import jax
import jax.numpy as jnp
from jax import lax
from jax.experimental import pallas as pl
from jax.experimental.pallas import tpu as pltpu
from jax.experimental.pallas import tpu_sc as plsc

D = 128
CHUNK = 80
NSUB = 16
NB = 2
EDGE_ALIGN = NSUB * CHUNK * 2 * NB


@functools.lru_cache(maxsize=None)
def _make_sc_kernel(n, e1, e2):
    mesh = plsc.VectorSubcoreMesh(core_axis_name="c", subcore_axis_name="s")
    per_tile1 = e1 // NSUB
    per_tile2 = e2 // NSUB
    n_iter1 = per_tile1 // CHUNK
    n_iter2 = per_tile2 // CHUNK
    rows_out = -(-n // (NSUB * 8)) * 8
    n_acc = rows_out * NSUB

    out_sds = jax.ShapeDtypeStruct((n_acc, D), jnp.float32)

    @functools.partial(
        pl.kernel,
        mesh=mesh,
        out_type=[out_sds, out_sds],
        scratch_types=[
            pltpu.VMEM((2 * NB, CHUNK), jnp.int32),
            pltpu.VMEM((2 * NB, CHUNK), jnp.int32),
            pltpu.VMEM((2 * NB, CHUNK, D), jnp.float32),
            pltpu.VMEM_SHARED((n_acc, D), jnp.float32),
            pltpu.SemaphoreType.DMA((2 * NB,)),
            pltpu.SemaphoreType.DMA((2 * NB,)),
            pltpu.SemaphoreType.DMA((2 * NB,)),
        ],
    )
    def k(x_hbm, s1_hbm, d1_hbm, s2_hbm, d2_hbm, z_hbm, o1_hbm, o2_hbm,
          src_v, dst_v, rows_v, acc, isem, gsem, ssem):
        sid = lax.axis_index("s")
        cid = lax.axis_index("c")

        row0 = pl.multiple_of(sid * rows_out, 8)
        pltpu.sync_copy(z_hbm.at[pl.ds(row0, rows_out)],
                        acc.at[pl.ds(row0, rows_out)])
        plsc.subcore_barrier()

        def process(s_hbm, d_hbm, per_tile, n_iter):
            base = sid * per_tile

            def idx_start(c, u):
                e0 = base + c * CHUNK
                pltpu.async_copy(s_hbm.at[pl.ds(e0, CHUNK)],
                                 src_v.at[u], isem.at[u])
                pltpu.async_copy(d_hbm.at[pl.ds(e0, CHUNK)],
                                 dst_v.at[u], isem.at[u])

            def idx_wait(u):
                pltpu.make_async_copy(s_hbm.at[pl.ds(0, CHUNK)],
                                      src_v.at[u], isem.at[u]).wait()
                pltpu.make_async_copy(d_hbm.at[pl.ds(0, CHUNK)],
                                      dst_v.at[u], isem.at[u]).wait()

            def gather_wait(u):
                pltpu.make_async_copy(x_hbm.at[pl.ds(0, CHUNK)],
                                      rows_v.at[u], gsem.at[u]).wait()

            def scatter_wait(u):
                pltpu.make_async_copy(z_hbm.at[pl.ds(0, CHUNK)],
                                      rows_v.at[u], ssem.at[u]).wait()

            for b in range(NB):
                idx_start(b, b)

            @pl.loop(0, n_iter, step=2 * NB)
            def _(r):
                for S in (0, 1):
                    cb = r + S * NB
                    for b in range(NB):
                        u = S * NB + b
                        idx_wait(u)
                        pltpu.async_copy(x_hbm.at[src_v.at[u]],
                                         rows_v.at[u], gsem.at[u])

                    for b in range(NB):
                        u = S * NB + b
                        gather_wait(u)
                        pltpu.async_copy(rows_v.at[u], acc.at[dst_v.at[u]],
                                         ssem.at[u], add=True)

                    for b in range(NB):
                        u2 = (1 - S) * NB + b
                        c2 = cb + NB + b
                        if S == 0:
                            @pl.when(r > 0)
                            def _():
                                scatter_wait(u2)
                        else:
                            scatter_wait(u2)

                        @pl.when(c2 < n_iter)
                        def _():
                            idx_start(c2, u2)

            for b in range(NB):
                scatter_wait(NB + b)

        @pl.when(cid == 0)
        def _():
            process(s1_hbm, d1_hbm, per_tile1, n_iter1)

        @pl.when(cid == 1)
        def _():
            process(s2_hbm, d2_hbm, per_tile2, n_iter2)

        plsc.subcore_barrier()

        sl = pl.ds(row0, rows_out)

        @pl.when(cid == 0)
        def _():
            pltpu.sync_copy(acc.at[sl], o1_hbm.at[sl])

        @pl.when(cid == 1)
        def _():
            pltpu.sync_copy(acc.at[sl], o2_hbm.at[sl])

    return k


def _pad_edges(src, dst, n, n_acc):
    e = src.shape[0]
    e_pad = -(-e // EDGE_ALIGN) * EDGE_ALIGN
    if e_pad != e:
        pad = e_pad - e
        src = jnp.concatenate([src, jnp.zeros((pad,), src.dtype)])
        trash = n + jnp.arange(pad, dtype=dst.dtype) % (n_acc - n)
        dst = jnp.concatenate([dst, trash])
    return src, dst


def kernel(x, adj_t, adj_t2):
    n = x.shape[0]
    n_acc = -(-n // (NSUB * 8)) * 8 * NSUB
    s1, d1 = _pad_edges(adj_t[0], adj_t[1], n, n_acc)
    s2, d2 = _pad_edges(adj_t2[0], adj_t2[1], n, n_acc)
    zeros = jnp.zeros((n_acc, D), jnp.float32)
    k = _make_sc_kernel(n, s1.shape[0], s2.shape[0])
    x1, x2 = k(x, s1, d1, s2, d2, zeros)
    return jnp.concatenate([x1[:n], x2[:n]], axis=1)
